# baseline (speedup 1.0000x reference)
import functools

import jax
import jax.numpy as jnp
from jax import lax
from jax.experimental import pallas as pl
from jax.experimental.pallas import tpu as pltpu
from jax.experimental.pallas import tpu_sc as plsc

NC = 2
NS = 16
L = 16
NW = NC * NS
ROWS = 128
COLS = 100000
RPW = ROWS // NW
NVEC = COLS // L

_mesh = plsc.VectorSubcoreMesh(core_axis_name="c", subcore_axis_name="s")


@functools.partial(
    pl.kernel,
    out_type=jax.ShapeDtypeStruct((NW, L), jnp.int32),
    mesh=_mesh,
    scratch_types=[
        pltpu.VMEM((COLS,), jnp.float32),
        pltpu.VMEM((L,), jnp.int32),
        pltpu.SemaphoreType.DMA,
    ],
)
def _argmax_sc(x_hbm, out_hbm, buf, res, sem):
    wid = lax.axis_index("s") * NC + lax.axis_index("c")
    iota = lax.iota(jnp.int32, L)
    res_vec = jnp.zeros((L,), jnp.int32)

    for j in range(RPW):
        row = wid * RPW + j
        pltpu.async_copy(x_hbm.at[row], buf, sem).wait()

        def body(i, carry):
            vmax, vidx, cur = carry
            v = buf[pl.ds(i * L, L)]
            pred = v > vmax
            vmax = jnp.where(pred, v, vmax)
            vidx = jnp.where(pred, cur, vidx)
            return vmax, vidx, cur + L

        init = (jnp.full((L,), -jnp.inf, jnp.float32), jnp.zeros((L,), jnp.int32), iota)
        vmax, vidx, _ = lax.fori_loop(0, NVEC, body, init, unroll=10)

        best_v = vmax[0]
        best_i = vidx[0]
        for k in range(1, L):
            pv = vmax[k]
            pi = vidx[k]
            pred = (pv > best_v) | ((pv == best_v) & (pi < best_i))
            best_v = jnp.where(pred, pv, best_v)
            best_i = jnp.where(pred, pi, best_i)
        res_vec = jnp.where(iota == j, best_i, res_vec)

    res[...] = res_vec
    pltpu.sync_copy(res, out_hbm.at[wid])


def kernel(m_logits):
    out = _argmax_sc(m_logits)
    return out[:, :RPW].reshape(ROWS, 1)
